# baseline (speedup 1.0000x reference)
import functools

import jax
import jax.numpy as jnp
from jax import lax
from jax.experimental import pallas as pl
from jax.experimental.pallas import tpu as pltpu
from jax.experimental.pallas import tpu_sc as plsc

NC = 2
NS = 16
CH = 128


def _mesh():
    return plsc.VectorSubcoreMesh(core_axis_name="c", subcore_axis_name="s")


def _sc_kernel(body, n, feat, extra_scratch):
    return functools.partial(
        pl.kernel,
        out_type=[jax.ShapeDtypeStruct((n, feat), jnp.float32),
                  jax.ShapeDtypeStruct((n, feat), jnp.float32)],
        mesh=_mesh(),
        compiler_params=pltpu.CompilerParams(use_tc_tiling_on_sc=False),
        scratch_types=extra_scratch,
    )(body)


def _make_edge_agg(n, feat, cpw, rem):
    rpt = n // NS
    R = 4 if feat <= 16 else 3
    ni = cpw // R
    lo = cpw - ni * R

    def body(table, eiv, zrows, out0, out1, ei_t, ex_t, *rest):
        bufs = rest[0:R]
        acc = rest[R]
        tbl_s = rest[R + 1]
        gs = rest[R + 2:R + 2 + R]
        ss = rest[R + 2 + R:R + 2 + 2 * R]
        c = lax.axis_index("c")
        sub = lax.axis_index("s")
        wid = c * NS + sub
        pltpu.sync_copy(zrows.at[pl.ds(sub * rpt, rpt)],
                        acc.at[pl.ds(sub * rpt, rpt)])
        pltpu.sync_copy(table.at[pl.ds(sub * rpt, rpt)],
                        tbl_s.at[pl.ds(sub * rpt, rpt)])
        pltpu.sync_copy(eiv.at[pl.ds(wid * cpw, cpw)], ei_t)
        if rem:
            pltpu.sync_copy(
                eiv.at[cpw * NC * NS + jnp.minimum(wid, rem - 1)], ex_t)
        plsc.subcore_barrier()

        for t in range(R):
            pltpu.async_copy(tbl_s.at[ei_t.at[t, 0]], bufs[t], gs[t])

        def step(i, carry):
            j0 = R * i
            for t in range(R):
                pltpu.make_async_copy(
                    tbl_s.at[ei_t.at[j0 + t, 0]], bufs[t], gs[t]).wait()
                pltpu.async_copy(
                    bufs[t], acc.at[ei_t.at[j0 + t, 1]], ss[t], add=True)
            for t in range(R):
                pltpu.make_async_copy(
                    bufs[t], acc.at[ei_t.at[j0 + t, 1]], ss[t]).wait()
                pltpu.async_copy(
                    tbl_s.at[ei_t.at[j0 + R + t, 0]], bufs[t], gs[t])
            return carry

        lax.fori_loop(0, ni - 1, step, 0)
        j0 = R * (ni - 1)
        for t in range(R):
            pltpu.make_async_copy(
                tbl_s.at[ei_t.at[j0 + t, 0]], bufs[t], gs[t]).wait()
            pltpu.async_copy(
                bufs[t], acc.at[ei_t.at[j0 + t, 1]], ss[t], add=True)
        for t in range(R):
            pltpu.make_async_copy(
                bufs[t], acc.at[ei_t.at[j0 + t, 1]], ss[t]).wait()

        for t in range(lo):
            j = R * ni + t
            pltpu.async_copy(tbl_s.at[ei_t.at[j, 0]], bufs[t], gs[t])
        for t in range(lo):
            j = R * ni + t
            pltpu.make_async_copy(
                tbl_s.at[ei_t.at[j, 0]], bufs[t], gs[t]).wait()
            pltpu.async_copy(
                bufs[t], acc.at[ei_t.at[j, 1]], ss[t], add=True)
        for t in range(lo):
            j = R * ni + t
            pltpu.make_async_copy(
                bufs[t], acc.at[ei_t.at[j, 1]], ss[t]).wait()

        if rem:
            @pl.when(wid < rem)
            def _():
                pltpu.async_copy(tbl_s.at[ex_t.at[0]], bufs[0], gs[0])
                pltpu.make_async_copy(
                    tbl_s.at[ex_t.at[0]], bufs[0], gs[0]).wait()
                pltpu.async_copy(bufs[0], acc.at[ex_t.at[1]], ss[0], add=True)
                pltpu.make_async_copy(
                    bufs[0], acc.at[ex_t.at[1]], ss[0]).wait()

        plsc.subcore_barrier()

        @pl.when(c == 0)
        def _():
            pltpu.sync_copy(acc.at[pl.ds(sub * rpt, rpt)],
                            out0.at[pl.ds(sub * rpt, rpt)])

        @pl.when(c == 1)
        def _():
            pltpu.sync_copy(acc.at[pl.ds(sub * rpt, rpt)],
                            out1.at[pl.ds(sub * rpt, rpt)])

    return _sc_kernel(body, n, feat, [
        pltpu.VMEM((cpw, 2, CH), jnp.int32),
        pltpu.VMEM((2, CH), jnp.int32),
    ] + [pltpu.VMEM((CH, feat), jnp.float32) for _ in range(R)] + [
        pltpu.VMEM_SHARED((n, feat), jnp.float32),
        pltpu.VMEM_SHARED((n, feat), jnp.float32),
    ] + [pltpu.SemaphoreType.DMA for _ in range(2 * R)])


def _make_deg(n, feat, cpw, rem):
    rpt = n // NS
    R = 2
    ni = cpw // R

    def body(eiv, ones_hbm, zrows, out0, out1, ei_t, ex_t, ones_v, acc,
             s0, s1):
        ss = (s0, s1)
        c = lax.axis_index("c")
        sub = lax.axis_index("s")
        wid = c * NS + sub
        pltpu.sync_copy(zrows.at[pl.ds(sub * rpt, rpt)],
                        acc.at[pl.ds(sub * rpt, rpt)])
        pltpu.sync_copy(eiv.at[pl.ds(wid * cpw, cpw)], ei_t)
        if rem:
            pltpu.sync_copy(
                eiv.at[cpw * NC * NS + jnp.minimum(wid, rem - 1)], ex_t)
        pltpu.sync_copy(ones_hbm, ones_v)
        plsc.subcore_barrier()

        for t in range(R):
            pltpu.async_copy(ones_v, acc.at[ei_t.at[t, 1]], ss[t], add=True)

        def step(i, carry):
            j0 = R * i
            for t in range(R):
                pltpu.make_async_copy(
                    ones_v, acc.at[ei_t.at[j0 + t, 1]], ss[t]).wait()
                pltpu.async_copy(
                    ones_v, acc.at[ei_t.at[j0 + R + t, 1]], ss[t], add=True)
            return carry

        lax.fori_loop(0, ni - 1, step, 0)
        j0 = R * (ni - 1)
        for t in range(R):
            pltpu.make_async_copy(
                ones_v, acc.at[ei_t.at[j0 + t, 1]], ss[t]).wait()

        if rem:
            @pl.when(wid < rem)
            def _():
                pltpu.async_copy(ones_v, acc.at[ex_t.at[1]], s0, add=True)
                pltpu.make_async_copy(ones_v, acc.at[ex_t.at[1]], s0).wait()

        plsc.subcore_barrier()

        @pl.when(c == 0)
        def _():
            pltpu.sync_copy(acc.at[pl.ds(sub * rpt, rpt)],
                            out0.at[pl.ds(sub * rpt, rpt)])

        @pl.when(c == 1)
        def _():
            pltpu.sync_copy(acc.at[pl.ds(sub * rpt, rpt)],
                            out1.at[pl.ds(sub * rpt, rpt)])

    return _sc_kernel(body, n, feat, [
        pltpu.VMEM((cpw, 2, CH), jnp.int32),
        pltpu.VMEM((2, CH), jnp.int32),
        pltpu.VMEM((CH, feat), jnp.float32),
        pltpu.VMEM_SHARED((n, feat), jnp.float32),
        pltpu.SemaphoreType.DMA,
        pltpu.SemaphoreType.DMA,
    ])


def _mm1(x_ref, w1_ref, h_ref):
    h_ref[...] = jnp.dot(x_ref[...], w1_ref[...],
                         preferred_element_type=jnp.float32)


def _scale1(h_ref, d0_ref, d1_ref, hs_ref, dinv_ref):
    deg = d0_ref[:, 0:1] + d1_ref[:, 0:1] + 1.0
    dinv = lax.rsqrt(deg)
    hs_ref[...] = h_ref[...] * dinv
    dinv_ref[...] = dinv


def _dense2(p0_ref, p1_ref, hs1_ref, dinv_ref, b1_ref, w2_ref, hs2_ref):
    dinv = dinv_ref[...]
    out1 = dinv * (p0_ref[...] + p1_ref[...] + hs1_ref[...]) + b1_ref[...]
    h = jnp.maximum(out1, 0.0)
    z = jnp.dot(h, w2_ref[...], preferred_element_type=jnp.float32)
    hs2_ref[...] = z * dinv


def _dense3(p0_ref, p1_ref, hs2_ref, dinv_ref, b2_ref, out_ref):
    dinv = dinv_ref[...]
    logits = dinv * (p0_ref[...] + p1_ref[...] + hs2_ref[...]) + b2_ref[...]
    ncls = out_ref.shape[1]
    mask = lax.broadcasted_iota(jnp.int32, logits.shape, 1) < ncls
    m = jnp.max(jnp.where(mask, logits, -1e30), axis=1, keepdims=True)
    e = jnp.where(mask, jnp.exp(logits - m), 0.0)
    lse = m + jnp.log(jnp.sum(e, axis=1, keepdims=True))
    out_ref[...] = (logits - lse)[:, :ncls]


def _tc_call(fn, out_shapes, *args):
    return pl.pallas_call(
        fn,
        out_shape=[jax.ShapeDtypeStruct(s, jnp.float32) for s in out_shapes],
    )(*args)


def kernel(x, edge_index, W1, b1, W2, b2):
    n, _ = x.shape
    h_dim = W1.shape[1]
    n_cls = W2.shape[1]
    e = edge_index.shape[1]
    nw = NC * NS

    f2 = 16
    g = e // CH
    cpw = g // nw
    rem = g - cpw * nw

    eiv = edge_index.reshape(2, g, CH).transpose(1, 0, 2)

    zrows64 = jnp.zeros((n, h_dim), jnp.float32)
    zrows16 = jnp.zeros((n, f2), jnp.float32)
    ones16 = jnp.ones((CH, f2), jnp.float32)

    deg0, deg1 = _make_deg(n, f2, cpw, rem)(eiv, ones16, zrows16)
    (h1,) = _tc_call(_mm1, [(n, h_dim)], x, W1)

    hs1, dinv = _tc_call(
        _scale1, [(n, h_dim), (n, 1)], h1, deg0, deg1)

    agg10, agg11 = _make_edge_agg(n, h_dim, cpw, rem)(hs1, eiv, zrows64)

    w2p = jnp.pad(W2, ((0, 0), (0, f2 - n_cls)))
    (hs2,) = _tc_call(
        _dense2, [(n, f2)],
        agg10, agg11, hs1, dinv, b1.reshape(1, h_dim), w2p)

    agg20, agg21 = _make_edge_agg(n, f2, cpw, rem)(hs2, eiv, zrows16)

    b2p = jnp.pad(b2, (0, f2 - n_cls)).reshape(1, f2)
    (outp,) = _tc_call(
        _dense3, [(n, n_cls)],
        agg20, agg21, hs2, dinv, b2p)
    return outp


# speedup vs baseline: 1.0768x; 1.0768x over previous
import functools

import jax
import jax.numpy as jnp
from jax import lax
from jax.experimental import pallas as pl
from jax.experimental.pallas import tpu as pltpu
from jax.experimental.pallas import tpu_sc as plsc

NC = 2
NS = 16
CH = 128


def _mesh():
    return plsc.VectorSubcoreMesh(core_axis_name="c", subcore_axis_name="s")


def _sc_kernel(body, n, feat, extra_scratch):
    return functools.partial(
        pl.kernel,
        out_type=[jax.ShapeDtypeStruct((n, feat), jnp.float32),
                  jax.ShapeDtypeStruct((n, feat), jnp.float32)],
        mesh=_mesh(),
        compiler_params=pltpu.CompilerParams(use_tc_tiling_on_sc=False),
        scratch_types=extra_scratch,
    )(body)


def _make_edge_agg(n, feat, cpw, rem):
    rpt = n // NS
    R = 4 if feat <= 16 else 2
    ni = cpw // R
    lo = cpw - ni * R

    def body(table, eiv, zrows, out0, out1, ei_t, ex_t, *rest):
        bufs = rest[0:R]
        acc = rest[R]
        tbl_s = rest[R + 1]
        gs = rest[R + 2:R + 2 + R]
        ss = rest[R + 2 + R:R + 2 + 2 * R]
        c = lax.axis_index("c")
        sub = lax.axis_index("s")
        wid = c * NS + sub
        pltpu.sync_copy(zrows.at[pl.ds(sub * rpt, rpt)],
                        acc.at[pl.ds(sub * rpt, rpt)])
        pltpu.sync_copy(table.at[pl.ds(sub * rpt, rpt)],
                        tbl_s.at[pl.ds(sub * rpt, rpt)])
        pltpu.sync_copy(eiv.at[pl.ds(wid * cpw, cpw)], ei_t)
        if rem:
            pltpu.sync_copy(
                eiv.at[cpw * NC * NS + jnp.minimum(wid, rem - 1)], ex_t)
        plsc.subcore_barrier()

        for t in range(R):
            pltpu.async_copy(tbl_s.at[ei_t.at[t, 0]], bufs[t], gs[t])

        def step(i, carry):
            j0 = R * i
            for t in range(R):
                pltpu.make_async_copy(
                    tbl_s.at[ei_t.at[j0 + t, 0]], bufs[t], gs[t]).wait()
                pltpu.async_copy(
                    bufs[t], acc.at[ei_t.at[j0 + t, 1]], ss[t], add=True)
            for t in range(R):
                pltpu.make_async_copy(
                    bufs[t], acc.at[ei_t.at[j0 + t, 1]], ss[t]).wait()
                pltpu.async_copy(
                    tbl_s.at[ei_t.at[j0 + R + t, 0]], bufs[t], gs[t])
            return carry

        lax.fori_loop(0, ni - 1, step, 0)
        j0 = R * (ni - 1)
        for t in range(R):
            pltpu.make_async_copy(
                tbl_s.at[ei_t.at[j0 + t, 0]], bufs[t], gs[t]).wait()
            pltpu.async_copy(
                bufs[t], acc.at[ei_t.at[j0 + t, 1]], ss[t], add=True)
        for t in range(R):
            pltpu.make_async_copy(
                bufs[t], acc.at[ei_t.at[j0 + t, 1]], ss[t]).wait()

        for t in range(lo):
            j = R * ni + t
            pltpu.async_copy(tbl_s.at[ei_t.at[j, 0]], bufs[t], gs[t])
        for t in range(lo):
            j = R * ni + t
            pltpu.make_async_copy(
                tbl_s.at[ei_t.at[j, 0]], bufs[t], gs[t]).wait()
            pltpu.async_copy(
                bufs[t], acc.at[ei_t.at[j, 1]], ss[t], add=True)
        for t in range(lo):
            j = R * ni + t
            pltpu.make_async_copy(
                bufs[t], acc.at[ei_t.at[j, 1]], ss[t]).wait()

        if rem:
            @pl.when(wid < rem)
            def _():
                pltpu.async_copy(tbl_s.at[ex_t.at[0]], bufs[0], gs[0])
                pltpu.make_async_copy(
                    tbl_s.at[ex_t.at[0]], bufs[0], gs[0]).wait()
                pltpu.async_copy(bufs[0], acc.at[ex_t.at[1]], ss[0], add=True)
                pltpu.make_async_copy(
                    bufs[0], acc.at[ex_t.at[1]], ss[0]).wait()

        plsc.subcore_barrier()

        @pl.when(c == 0)
        def _():
            pltpu.sync_copy(acc.at[pl.ds(sub * rpt, rpt)],
                            out0.at[pl.ds(sub * rpt, rpt)])

        @pl.when(c == 1)
        def _():
            pltpu.sync_copy(acc.at[pl.ds(sub * rpt, rpt)],
                            out1.at[pl.ds(sub * rpt, rpt)])

    return _sc_kernel(body, n, feat, [
        pltpu.VMEM((cpw, 2, CH), jnp.int32),
        pltpu.VMEM((2, CH), jnp.int32),
    ] + [pltpu.VMEM((CH, feat), jnp.float32) for _ in range(R)] + [
        pltpu.VMEM_SHARED((n, feat), jnp.float32),
        pltpu.VMEM_SHARED((n, feat), jnp.float32),
    ] + [pltpu.SemaphoreType.DMA for _ in range(2 * R)])


def _make_deg(n, feat, cpw, rem):
    rpt = n // NS
    R = 2
    ni = cpw // R

    def body(eiv, ones_hbm, zrows, out0, out1, ei_t, ex_t, ones_v, acc,
             s0, s1):
        ss = (s0, s1)
        c = lax.axis_index("c")
        sub = lax.axis_index("s")
        wid = c * NS + sub
        pltpu.sync_copy(zrows.at[pl.ds(sub * rpt, rpt)],
                        acc.at[pl.ds(sub * rpt, rpt)])
        pltpu.sync_copy(eiv.at[pl.ds(wid * cpw, cpw)], ei_t)
        if rem:
            pltpu.sync_copy(
                eiv.at[cpw * NC * NS + jnp.minimum(wid, rem - 1)], ex_t)
        pltpu.sync_copy(ones_hbm, ones_v)
        plsc.subcore_barrier()

        for t in range(R):
            pltpu.async_copy(ones_v, acc.at[ei_t.at[t, 1]], ss[t], add=True)

        def step(i, carry):
            j0 = R * i
            for t in range(R):
                pltpu.make_async_copy(
                    ones_v, acc.at[ei_t.at[j0 + t, 1]], ss[t]).wait()
                pltpu.async_copy(
                    ones_v, acc.at[ei_t.at[j0 + R + t, 1]], ss[t], add=True)
            return carry

        lax.fori_loop(0, ni - 1, step, 0)
        j0 = R * (ni - 1)
        for t in range(R):
            pltpu.make_async_copy(
                ones_v, acc.at[ei_t.at[j0 + t, 1]], ss[t]).wait()

        if rem:
            @pl.when(wid < rem)
            def _():
                pltpu.async_copy(ones_v, acc.at[ex_t.at[1]], s0, add=True)
                pltpu.make_async_copy(ones_v, acc.at[ex_t.at[1]], s0).wait()

        plsc.subcore_barrier()

        @pl.when(c == 0)
        def _():
            pltpu.sync_copy(acc.at[pl.ds(sub * rpt, rpt)],
                            out0.at[pl.ds(sub * rpt, rpt)])

        @pl.when(c == 1)
        def _():
            pltpu.sync_copy(acc.at[pl.ds(sub * rpt, rpt)],
                            out1.at[pl.ds(sub * rpt, rpt)])

    return _sc_kernel(body, n, feat, [
        pltpu.VMEM((cpw, 2, CH), jnp.int32),
        pltpu.VMEM((2, CH), jnp.int32),
        pltpu.VMEM((CH, feat), jnp.float32),
        pltpu.VMEM_SHARED((n, feat), jnp.float32),
        pltpu.SemaphoreType.DMA,
        pltpu.SemaphoreType.DMA,
    ])


def _mm1(x_ref, w1_ref, h_ref):
    h_ref[...] = jnp.dot(x_ref[...], w1_ref[...],
                         preferred_element_type=jnp.float32)


def _scale1(h_ref, d0_ref, d1_ref, hs_ref, dinv_ref):
    deg = d0_ref[:, 0:1] + d1_ref[:, 0:1] + 1.0
    dinv = lax.rsqrt(deg)
    hs_ref[...] = h_ref[...] * dinv
    dinv_ref[...] = dinv


def _dense2(p0_ref, p1_ref, hs1_ref, dinv_ref, b1_ref, w2_ref, hs2_ref):
    dinv = dinv_ref[...]
    out1 = dinv * (p0_ref[...] + p1_ref[...] + hs1_ref[...]) + b1_ref[...]
    h = jnp.maximum(out1, 0.0)
    z = jnp.dot(h, w2_ref[...], preferred_element_type=jnp.float32)
    hs2_ref[...] = z * dinv


def _dense3(p0_ref, p1_ref, hs2_ref, dinv_ref, b2_ref, out_ref):
    dinv = dinv_ref[...]
    logits = dinv * (p0_ref[...] + p1_ref[...] + hs2_ref[...]) + b2_ref[...]
    ncls = out_ref.shape[1]
    mask = lax.broadcasted_iota(jnp.int32, logits.shape, 1) < ncls
    m = jnp.max(jnp.where(mask, logits, -1e30), axis=1, keepdims=True)
    e = jnp.where(mask, jnp.exp(logits - m), 0.0)
    lse = m + jnp.log(jnp.sum(e, axis=1, keepdims=True))
    out_ref[...] = (logits - lse)[:, :ncls]


def _tc_call(fn, out_shapes, *args):
    return pl.pallas_call(
        fn,
        out_shape=[jax.ShapeDtypeStruct(s, jnp.float32) for s in out_shapes],
    )(*args)


def kernel(x, edge_index, W1, b1, W2, b2):
    n, _ = x.shape
    h_dim = W1.shape[1]
    n_cls = W2.shape[1]
    e = edge_index.shape[1]
    nw = NC * NS

    f2 = 16
    g = e // CH
    cpw = g // nw
    rem = g - cpw * nw

    eiv = edge_index.reshape(2, g, CH).transpose(1, 0, 2)

    zrows64 = jnp.zeros((n, h_dim), jnp.float32)
    zrows16 = jnp.zeros((n, f2), jnp.float32)
    ones16 = jnp.ones((CH, f2), jnp.float32)

    deg0, deg1 = _make_deg(n, f2, cpw, rem)(eiv, ones16, zrows16)
    (h1,) = _tc_call(_mm1, [(n, h_dim)], x, W1)

    hs1, dinv = _tc_call(
        _scale1, [(n, h_dim), (n, 1)], h1, deg0, deg1)

    agg10, agg11 = _make_edge_agg(n, h_dim, cpw, rem)(hs1, eiv, zrows64)

    w2p = jnp.pad(W2, ((0, 0), (0, f2 - n_cls)))
    (hs2,) = _tc_call(
        _dense2, [(n, f2)],
        agg10, agg11, hs1, dinv, b1.reshape(1, h_dim), w2p)

    agg20, agg21 = _make_edge_agg(n, f2, cpw, rem)(hs2, eiv, zrows16)

    b2p = jnp.pad(b2, (0, f2 - n_cls)).reshape(1, f2)
    (outp,) = _tc_call(
        _dense3, [(n, n_cls)],
        agg20, agg21, hs2, dinv, b2p)
    return outp
